# Initial kernel scaffold; baseline (speedup 1.0000x reference)
import jax
import jax.numpy as jnp
from jax.experimental import pallas as pl


def kernel(node_features, senders, receivers, edge_features, W_kernel, W_bias, We_kernel, We_bias):
    raise NotImplementedError("write your pallas kernel here")


# speedup vs baseline: 6.3133x; 6.3133x over previous
import dataclasses
import functools
import jax
import jax.numpy as jnp
from jax import lax
from jax.experimental import pallas as pl
from jax.experimental.pallas import tpu as pltpu
from jax.experimental.pallas import tpu_sc as plsc

N = 10000
E = 160000
D = 256
DE = 16

NC = 2
NS = 16
CH = 128
N_PAD = 10240
E_PAD = 163840
N_CHUNKS = E_PAD // CH
CPS = N_CHUNKS // NS
ROWS = N_PAD // NS
HALF = D // 2
GW = 32
BLK = 8

_mesh = plsc.VectorSubcoreMesh(core_axis_name="c", subcore_axis_name="s",
                               num_cores=NC, num_subcores=NS)

_sc_params = pltpu.CompilerParams()
if "needs_layout_passes" in pltpu.CompilerParams.__dataclass_fields__:
    _sc_params = dataclasses.replace(_sc_params, needs_layout_passes=False)


@functools.partial(
    pl.kernel,
    out_type=jax.ShapeDtypeStruct((NC, N_PAD, 16), jnp.float32),
    mesh=_mesh,
    scratch_types=[
        pltpu.VMEM_SHARED((N_PAD, 16), jnp.float32),
        pltpu.VMEM((CPS // 2, CH), jnp.int32),
        pltpu.VMEM((CH, 16), jnp.float32),
        pltpu.VMEM((CH, 16), jnp.float32),
    ],
)
def _sc_hist(recv_hbm, hist_out, hist_sh, ridx, onehot, zbuf):
    c = lax.axis_index("c")
    s = lax.axis_index("s")
    w = s * NC + c
    zeros16 = jnp.zeros((16,), jnp.float32)
    one0 = jnp.where(lax.iota(jnp.int32, 16) == 0,
                     jnp.float32(1.0), jnp.float32(0.0))

    @pl.loop(0, CH)
    def _(i):
        onehot[i, :] = one0
        zbuf[i, :] = zeros16

    @pl.loop(0, ROWS // CH)
    def _(k):
        pltpu.sync_copy(zbuf, hist_sh.at[pl.ds(s * ROWS + k * CH, CH)])

    pltpu.sync_copy(recv_hbm.at[pl.ds(w * (CPS // 2), CPS // 2)], ridx)
    plsc.subcore_barrier()

    @pl.loop(0, CPS // 2)
    def _(j):
        pltpu.sync_copy(onehot, hist_sh.at[ridx.at[j]], add=True)

    plsc.subcore_barrier()
    pltpu.sync_copy(hist_sh.at[pl.ds(s * ROWS, ROWS)],
                    hist_out.at[c].at[pl.ds(s * ROWS, ROWS)])


@functools.partial(
    pl.kernel,
    out_type=jax.ShapeDtypeStruct((NC, N_PAD, HALF), jnp.float32),
    mesh=_mesh,
    scratch_types=[
        pltpu.VMEM_SHARED((N_PAD, HALF), jnp.float32),
        pltpu.VMEM((BLK, CH), jnp.int32),
        pltpu.VMEM((BLK, CH), jnp.int32),
        pltpu.VMEM((CH, HALF), jnp.float32),
    ],
    compiler_params=_sc_params,
)
def _sc_edges(hs_hbm, send_hbm, recv_hbm,
              acc_out, acc_sh, sidx, ridx, hbuf):
    c = lax.axis_index("c")
    s = lax.axis_index("s")
    zeros16 = jnp.zeros((16,), jnp.float32)

    @pl.loop(0, CH)
    def _(i):
        @pl.loop(0, HALF, step=16)
        def _(q):
            hbuf[i, pl.ds(q, 16)] = zeros16

    @pl.loop(0, ROWS // CH)
    def _(k):
        pltpu.sync_copy(hbuf, acc_sh.at[pl.ds(s * ROWS + k * CH, CH)])

    plsc.subcore_barrier()

    @pl.loop(0, CPS // BLK)
    def _(b):
        pltpu.sync_copy(send_hbm.at[pl.ds(s * CPS + b * BLK, BLK)], sidx)
        pltpu.sync_copy(recv_hbm.at[pl.ds(s * CPS + b * BLK, BLK)], ridx)

        @pl.loop(0, BLK)
        def _(j):
            pltpu.sync_copy(hs_hbm.at[c].at[sidx.at[j]], hbuf)
            pltpu.sync_copy(hbuf, acc_sh.at[ridx.at[j]], add=True)

    plsc.subcore_barrier()
    pltpu.sync_copy(acc_sh.at[pl.ds(s * ROWS, ROWS)],
                    acc_out.at[c].at[pl.ds(s * ROWS, ROWS)])


CPW = N_CHUNKS // (NC * NS)


@functools.partial(
    pl.kernel,
    out_type=jax.ShapeDtypeStruct((NC, N_PAD, GW), jnp.float32),
    mesh=_mesh,
    scratch_types=[
        pltpu.VMEM_SHARED((N_PAD, GW), jnp.float32),
        pltpu.VMEM((BLK, CH), jnp.int32),
        pltpu.VMEM((BLK, CH), jnp.int32),
        pltpu.VMEM((CH,), jnp.float32),
        pltpu.VMEM((DE, CH), jnp.float32),
        pltpu.VMEM((CH, GW), jnp.float32),
    ],
    compiler_params=_sc_params,
)
def _sc_gfeat(send_hbm, recv_hbm, eft_hbm, rs_hbm,
              g_out, g_sh, sidx, ridx, rsbuf, eftv, gbuf):
    c = lax.axis_index("c")
    s = lax.axis_index("s")
    w = s * NC + c
    zeros16 = jnp.zeros((16,), jnp.float32)
    iota16 = lax.iota(jnp.int32, 16)

    @pl.loop(0, CH)
    def _(i):
        @pl.loop(0, GW, step=16)
        def _(q):
            gbuf[i, pl.ds(q, 16)] = zeros16

    @pl.loop(0, ROWS // CH)
    def _(k):
        pltpu.sync_copy(gbuf, g_sh.at[pl.ds(s * ROWS + k * CH, CH)])

    plsc.subcore_barrier()

    @pl.loop(0, CPW // BLK)
    def _(b):
        pltpu.sync_copy(send_hbm.at[pl.ds(w * CPW + b * BLK, BLK)], sidx)
        pltpu.sync_copy(recv_hbm.at[pl.ds(w * CPW + b * BLK, BLK)], ridx)

        @pl.loop(0, BLK)
        def _(j):
            jj = (w * CPW + b * BLK + j) * CH
            pltpu.sync_copy(eft_hbm.at[:, pl.ds(jj, CH)], eftv)
            pltpu.sync_copy(rs_hbm.at[sidx.at[j]], rsbuf)
            for g in range(CH // 16):
                rg = rsbuf[pl.ds(g * 16, 16)]
                rows = iota16 + g * 16
                for d in range(DE):
                    v = eftv[d, pl.ds(g * 16, 16)] * rg
                    plsc.store_scatter(
                        gbuf, [rows, jnp.full((16,), d, jnp.int32)], v)
                plsc.store_scatter(
                    gbuf, [rows, jnp.full((16,), DE, jnp.int32)], rg)
            pltpu.sync_copy(gbuf, g_sh.at[ridx.at[j]], add=True)

    plsc.subcore_barrier()
    pltpu.sync_copy(g_sh.at[pl.ds(s * ROWS, ROWS)],
                    g_out.at[c].at[pl.ds(s * ROWS, ROWS)])


def _tc_h_body(x_ref, w_ref, b_ref, h_ref):
    h_ref[...] = (
        jax.lax.dot_general(x_ref[...], w_ref[...], (((1,), (0,)), ((), ())),
                            precision=jax.lax.Precision.HIGHEST,
                            preferred_element_type=jnp.float32)
        + b_ref[...])


def _tc_scale_body(h_ref, hist_ref, hs_ref, rs_ref):
    deg = jnp.sum(hist_ref[...], axis=(0, 2)) + 1.0
    rs = lax.rsqrt(deg)
    hs = h_ref[...] * rs[:, None]
    hs_ref[0, ...] = hs[:, :HALF]
    hs_ref[1, ...] = hs[:, HALF:]
    rs_ref[...] = rs


def _tc_final_body(acc_ref, g_ref, hs_ref, rs_ref, we_ref, be_ref, out_ref):
    g = g_ref[0, ...] + g_ref[1, ...]
    ef_part = jax.lax.dot_general(
        g[:, :DE], we_ref[...], (((1,), (0,)), ((), ())),
        precision=jax.lax.Precision.HIGHEST,
        preferred_element_type=jnp.float32)
    hs = jnp.concatenate([hs_ref[0, ...], hs_ref[1, ...]], axis=1)
    acc = jnp.concatenate([acc_ref[0, ...], acc_ref[1, ...]], axis=1)
    bias_part = g[:, DE:DE + 1] * be_ref[...][None, :]
    out_ref[...] = rs_ref[...][:, None] * (hs + acc + ef_part + bias_part)


@jax.jit
def kernel(node_features, senders, receivers, edge_features,
           W_kernel, W_bias, We_kernel, We_bias):
    senders = senders.astype(jnp.int32)
    receivers = receivers.astype(jnp.int32)

    pad_e = E_PAD - E
    s2d = jnp.concatenate(
        [senders, jnp.full((pad_e,), N, jnp.int32)]).reshape(N_CHUNKS, CH)
    r2d = jnp.concatenate(
        [receivers, jnp.full((pad_e,), N, jnp.int32)]).reshape(N_CHUNKS, CH)
    eft = jnp.concatenate(
        [edge_features.T, jnp.zeros((DE, pad_e), jnp.float32)], axis=1)
    xpad = jnp.concatenate(
        [node_features, jnp.zeros((N_PAD - N, D), jnp.float32)])

    hist = _sc_hist(r2d)

    RB = 1024
    grid = (N_PAD // RB,)

    h = pl.pallas_call(
        _tc_h_body,
        grid=grid,
        in_specs=[pl.BlockSpec((RB, D), lambda i: (i, 0)),
                  pl.BlockSpec((D, D), lambda i: (0, 0)),
                  pl.BlockSpec((D,), lambda i: (0,))],
        out_specs=pl.BlockSpec((RB, D), lambda i: (i, 0)),
        out_shape=jax.ShapeDtypeStruct((N_PAD, D), jnp.float32),
    )(xpad, W_kernel, W_bias)

    hs2, rs = pl.pallas_call(
        _tc_scale_body,
        grid=grid,
        in_specs=[pl.BlockSpec((RB, D), lambda i: (i, 0)),
                  pl.BlockSpec((NC, RB, 16), lambda i: (0, i, 0))],
        out_specs=(pl.BlockSpec((NC, RB, HALF), lambda i: (0, i, 0)),
                   pl.BlockSpec((RB,), lambda i: (i,))),
        out_shape=(jax.ShapeDtypeStruct((NC, N_PAD, HALF), jnp.float32),
                   jax.ShapeDtypeStruct((N_PAD,), jnp.float32)),
    )(h, hist)

    acc2 = _sc_edges(hs2, s2d, r2d)
    g2 = _sc_gfeat(s2d, r2d, eft, rs)

    out = pl.pallas_call(
        _tc_final_body,
        grid=grid,
        in_specs=[pl.BlockSpec((NC, RB, HALF), lambda i: (0, i, 0)),
                  pl.BlockSpec((NC, RB, GW), lambda i: (0, i, 0)),
                  pl.BlockSpec((NC, RB, HALF), lambda i: (0, i, 0)),
                  pl.BlockSpec((RB,), lambda i: (i,)),
                  pl.BlockSpec((DE, D), lambda i: (0, 0)),
                  pl.BlockSpec((D,), lambda i: (0,))],
        out_specs=pl.BlockSpec((RB, D), lambda i: (i, 0)),
        out_shape=jax.ShapeDtypeStruct((N_PAD, D), jnp.float32),
    )(acc2, g2, hs2, rs, We_kernel, We_bias)

    return out[:N]


# speedup vs baseline: 7.3270x; 1.1606x over previous
import dataclasses
import functools
import jax
import jax.numpy as jnp
from jax import lax
from jax.experimental import pallas as pl
from jax.experimental.pallas import tpu as pltpu
from jax.experimental.pallas import tpu_sc as plsc

N = 10000
E = 160000
D = 256
DE = 16

NC = 2
NS = 16
CH = 128
N_PAD = 10240
E_PAD = 163840
N_CHUNKS = E_PAD // CH
CPS = N_CHUNKS // NS
ROWS = N_PAD // NS
HALF = D // 2
GW = 32
BLK = 8

_mesh = plsc.VectorSubcoreMesh(core_axis_name="c", subcore_axis_name="s",
                               num_cores=NC, num_subcores=NS)

_sc_params = pltpu.CompilerParams()
if "needs_layout_passes" in pltpu.CompilerParams.__dataclass_fields__:
    _sc_params = dataclasses.replace(_sc_params, needs_layout_passes=False)


@functools.partial(
    pl.kernel,
    out_type=jax.ShapeDtypeStruct((NC, N_PAD, 16), jnp.float32),
    mesh=_mesh,
    scratch_types=[
        pltpu.VMEM_SHARED((N_PAD, 16), jnp.float32),
        pltpu.VMEM((CPS // 2, CH), jnp.int32),
        pltpu.VMEM((CH, 16), jnp.float32),
        pltpu.VMEM((CH, 16), jnp.float32),
    ],
)
def _sc_hist(recv_hbm, hist_out, hist_sh, ridx, onehot, zbuf):
    c = lax.axis_index("c")
    s = lax.axis_index("s")
    w = s * NC + c
    zeros16 = jnp.zeros((16,), jnp.float32)
    one0 = jnp.where(lax.iota(jnp.int32, 16) == 0,
                     jnp.float32(1.0), jnp.float32(0.0))

    @pl.loop(0, CH)
    def _(i):
        onehot[i, :] = one0
        zbuf[i, :] = zeros16

    @pl.loop(0, ROWS // CH)
    def _(k):
        pltpu.sync_copy(zbuf, hist_sh.at[pl.ds(s * ROWS + k * CH, CH)])

    pltpu.sync_copy(recv_hbm.at[pl.ds(w * (CPS // 2), CPS // 2)], ridx)
    plsc.subcore_barrier()

    @pl.loop(0, CPS // 2)
    def _(j):
        pltpu.sync_copy(onehot, hist_sh.at[ridx.at[j]], add=True)

    plsc.subcore_barrier()
    pltpu.sync_copy(hist_sh.at[pl.ds(s * ROWS, ROWS)],
                    hist_out.at[c].at[pl.ds(s * ROWS, ROWS)])


@functools.partial(
    pl.kernel,
    out_type=jax.ShapeDtypeStruct((NC, N_PAD, HALF), jnp.float32),
    mesh=_mesh,
    scratch_types=[
        pltpu.VMEM_SHARED((N_PAD, HALF), jnp.float32),
        pltpu.VMEM((BLK, CH), jnp.int32),
        pltpu.VMEM((BLK, CH), jnp.int32),
        pltpu.VMEM((CH, HALF), jnp.float32),
        pltpu.VMEM((CH, HALF), jnp.float32),
        pltpu.SemaphoreType.DMA,
        pltpu.SemaphoreType.DMA,
        pltpu.SemaphoreType.DMA,
        pltpu.SemaphoreType.DMA,
    ],
    compiler_params=_sc_params,
)
def _sc_edges(hs_hbm, send_hbm, recv_hbm,
              acc_out, acc_sh, sidx, ridx, hbufA, hbufB,
              gsA, gsB, ssA, ssB):
    c = lax.axis_index("c")
    s = lax.axis_index("s")
    zeros16 = jnp.zeros((16,), jnp.float32)
    bufs = (hbufA, hbufB)
    gsems = (gsA, gsB)
    ssems = (ssA, ssB)

    @pl.loop(0, CH)
    def _(i):
        @pl.loop(0, HALF, step=16)
        def _(q):
            hbufA[i, pl.ds(q, 16)] = zeros16

    @pl.loop(0, ROWS // CH)
    def _(k):
        pltpu.sync_copy(hbufA, acc_sh.at[pl.ds(s * ROWS + k * CH, CH)])

    plsc.subcore_barrier()

    @pl.loop(0, CPS // BLK)
    def _(b):
        pltpu.sync_copy(send_hbm.at[pl.ds(s * CPS + b * BLK, BLK)], sidx)
        pltpu.sync_copy(recv_hbm.at[pl.ds(s * CPS + b * BLK, BLK)], ridx)

        g = {}
        sc = {}
        g[0] = pltpu.async_copy(hs_hbm.at[c].at[sidx.at[0]], bufs[0], gsA)
        g[1] = pltpu.async_copy(hs_hbm.at[c].at[sidx.at[1]], bufs[1], gsB)
        for j in range(BLK):
            p = j & 1
            g[j].wait()
            sc[j] = pltpu.async_copy(
                bufs[p], acc_sh.at[ridx.at[j]], ssems[p], add=True)
            if j + 2 < BLK:
                sc[j].wait()
                g[j + 2] = pltpu.async_copy(
                    hs_hbm.at[c].at[sidx.at[j + 2]], bufs[p], gsems[p])
        sc[BLK - 2].wait()
        sc[BLK - 1].wait()

    plsc.subcore_barrier()
    pltpu.sync_copy(acc_sh.at[pl.ds(s * ROWS, ROWS)],
                    acc_out.at[c].at[pl.ds(s * ROWS, ROWS)])


CPW = N_CHUNKS // (NC * NS)


@functools.partial(
    pl.kernel,
    out_type=jax.ShapeDtypeStruct((NC, N_PAD, GW), jnp.float32),
    mesh=_mesh,
    scratch_types=[
        pltpu.VMEM_SHARED((N_PAD, GW), jnp.float32),
        pltpu.VMEM((BLK, CH), jnp.int32),
        pltpu.VMEM((BLK, CH), jnp.int32),
        pltpu.VMEM((2, CH), jnp.float32),
        pltpu.VMEM((2, DE, CH), jnp.float32),
        pltpu.VMEM((CH, GW), jnp.float32),
        pltpu.VMEM((CH, GW), jnp.float32),
        pltpu.SemaphoreType.DMA,
        pltpu.SemaphoreType.DMA,
        pltpu.SemaphoreType.DMA,
        pltpu.SemaphoreType.DMA,
    ],
    compiler_params=_sc_params,
)
def _sc_gfeat(send_hbm, recv_hbm, eft_hbm, rs_hbm,
              g_out, g_sh, sidx, ridx, rsbuf, eftv, gbufA, gbufB,
              leA, leB, ssA, ssB):
    c = lax.axis_index("c")
    s = lax.axis_index("s")
    w = s * NC + c
    zeros16 = jnp.zeros((16,), jnp.float32)
    iota16 = lax.iota(jnp.int32, 16)
    gbufs = (gbufA, gbufB)
    lsems = (leA, leB)
    ssems = (ssA, ssB)

    @pl.loop(0, CH)
    def _(i):
        @pl.loop(0, GW, step=16)
        def _(q):
            gbufA[i, pl.ds(q, 16)] = zeros16
            gbufB[i, pl.ds(q, 16)] = zeros16

    @pl.loop(0, ROWS // CH)
    def _(k):
        pltpu.sync_copy(gbufA, g_sh.at[pl.ds(s * ROWS + k * CH, CH)])

    plsc.subcore_barrier()

    @pl.loop(0, CPW // BLK)
    def _(b):
        base = w * CPW + b * BLK
        pltpu.sync_copy(send_hbm.at[pl.ds(base, BLK)], sidx)
        pltpu.sync_copy(recv_hbm.at[pl.ds(base, BLK)], ridx)

        def load(j, p):
            e = pltpu.async_copy(
                eft_hbm.at[:, pl.ds((base + j) * CH, CH)], eftv.at[p],
                lsems[p])
            r = pltpu.async_copy(rs_hbm.at[sidx.at[j]], rsbuf.at[p],
                                 lsems[p])
            return e, r

        ld = {0: load(0, 0), 1: load(1, 1)}
        sc = {}
        for j in range(BLK):
            p = j & 1
            ld[j][0].wait()
            ld[j][1].wait()
            if j >= 2:
                sc[j - 2].wait()
            for g in range(CH // 16):
                rg = rsbuf[p, pl.ds(g * 16, 16)]
                rows = iota16 + g * 16
                for d in range(DE):
                    v = eftv[p, d, pl.ds(g * 16, 16)] * rg
                    plsc.store_scatter(
                        gbufs[p], [rows, jnp.full((16,), d, jnp.int32)], v)
                plsc.store_scatter(
                    gbufs[p], [rows, jnp.full((16,), DE, jnp.int32)], rg)
            sc[j] = pltpu.async_copy(
                gbufs[p], g_sh.at[ridx.at[j]], ssems[p], add=True)
            if j + 2 < BLK:
                ld[j + 2] = load(j + 2, p)
        sc[BLK - 2].wait()
        sc[BLK - 1].wait()

    plsc.subcore_barrier()
    pltpu.sync_copy(g_sh.at[pl.ds(s * ROWS, ROWS)],
                    g_out.at[c].at[pl.ds(s * ROWS, ROWS)])


def _tc_h_body(x_ref, w_ref, b_ref, h_ref):
    h_ref[...] = (
        jax.lax.dot_general(x_ref[...], w_ref[...], (((1,), (0,)), ((), ())),
                            precision=jax.lax.Precision.HIGHEST,
                            preferred_element_type=jnp.float32)
        + b_ref[...])


def _tc_scale_body(h_ref, hist_ref, hs_ref, rs_ref):
    deg = jnp.sum(hist_ref[...], axis=(0, 2)) + 1.0
    rs = lax.rsqrt(deg)
    hs = h_ref[...] * rs[:, None]
    hs_ref[0, ...] = hs[:, :HALF]
    hs_ref[1, ...] = hs[:, HALF:]
    rs_ref[...] = rs


def _tc_final_body(acc_ref, g_ref, hs_ref, rs_ref, we_ref, be_ref, out_ref):
    g = g_ref[0, ...] + g_ref[1, ...]
    ef_part = jax.lax.dot_general(
        g[:, :DE], we_ref[...], (((1,), (0,)), ((), ())),
        precision=jax.lax.Precision.HIGHEST,
        preferred_element_type=jnp.float32)
    hs = jnp.concatenate([hs_ref[0, ...], hs_ref[1, ...]], axis=1)
    acc = jnp.concatenate([acc_ref[0, ...], acc_ref[1, ...]], axis=1)
    bias_part = g[:, DE:DE + 1] * be_ref[...][None, :]
    out_ref[...] = rs_ref[...][:, None] * (hs + acc + ef_part + bias_part)


@jax.jit
def kernel(node_features, senders, receivers, edge_features,
           W_kernel, W_bias, We_kernel, We_bias):
    senders = senders.astype(jnp.int32)
    receivers = receivers.astype(jnp.int32)

    pad_e = E_PAD - E
    s2d = jnp.concatenate(
        [senders, jnp.full((pad_e,), N, jnp.int32)]).reshape(N_CHUNKS, CH)
    r2d = jnp.concatenate(
        [receivers, jnp.full((pad_e,), N, jnp.int32)]).reshape(N_CHUNKS, CH)
    eft = jnp.concatenate(
        [edge_features.T, jnp.zeros((DE, pad_e), jnp.float32)], axis=1)
    xpad = jnp.concatenate(
        [node_features, jnp.zeros((N_PAD - N, D), jnp.float32)])

    hist = _sc_hist(r2d)

    RB = 1024
    grid = (N_PAD // RB,)

    h = pl.pallas_call(
        _tc_h_body,
        grid=grid,
        in_specs=[pl.BlockSpec((RB, D), lambda i: (i, 0)),
                  pl.BlockSpec((D, D), lambda i: (0, 0)),
                  pl.BlockSpec((D,), lambda i: (0,))],
        out_specs=pl.BlockSpec((RB, D), lambda i: (i, 0)),
        out_shape=jax.ShapeDtypeStruct((N_PAD, D), jnp.float32),
    )(xpad, W_kernel, W_bias)

    hs2, rs = pl.pallas_call(
        _tc_scale_body,
        grid=grid,
        in_specs=[pl.BlockSpec((RB, D), lambda i: (i, 0)),
                  pl.BlockSpec((NC, RB, 16), lambda i: (0, i, 0))],
        out_specs=(pl.BlockSpec((NC, RB, HALF), lambda i: (0, i, 0)),
                   pl.BlockSpec((RB,), lambda i: (i,))),
        out_shape=(jax.ShapeDtypeStruct((NC, N_PAD, HALF), jnp.float32),
                   jax.ShapeDtypeStruct((N_PAD,), jnp.float32)),
    )(h, hist)

    acc2 = _sc_edges(hs2, s2d, r2d)
    g2 = _sc_gfeat(s2d, r2d, eft, rs)

    out = pl.pallas_call(
        _tc_final_body,
        grid=grid,
        in_specs=[pl.BlockSpec((NC, RB, HALF), lambda i: (0, i, 0)),
                  pl.BlockSpec((NC, RB, GW), lambda i: (0, i, 0)),
                  pl.BlockSpec((NC, RB, HALF), lambda i: (0, i, 0)),
                  pl.BlockSpec((RB,), lambda i: (i,)),
                  pl.BlockSpec((DE, D), lambda i: (0, 0)),
                  pl.BlockSpec((D,), lambda i: (0,))],
        out_specs=pl.BlockSpec((RB, D), lambda i: (i, 0)),
        out_shape=jax.ShapeDtypeStruct((N_PAD, D), jnp.float32),
    )(acc2, g2, hs2, rs, We_kernel, We_bias)

    return out[:N]
